# Initial kernel scaffold; baseline (speedup 1.0000x reference)
import jax
import jax.numpy as jnp
from jax.experimental import pallas as pl


def kernel(points):
    raise NotImplementedError("write your pallas kernel here")


# speedup vs baseline: 30.5261x; 30.5261x over previous
import jax
import jax.numpy as jnp
from jax.experimental import pallas as pl
from jax.experimental.pallas import tpu as pltpu

_B = 16
_N = 8192
_S = 2048


def _fps_body(x_ref, y_ref, z_ref, idx_ref, xo_ref, yo_ref, zo_ref, dist_ref):
    x = x_ref[...]
    y = y_ref[...]
    z = z_ref[...]
    lane = jax.lax.broadcasted_iota(jnp.int32, (_B, _N), 1)
    r = jax.lax.broadcasted_iota(jnp.int32, (_B, _B), 0)
    c = jax.lax.broadcasted_iota(jnp.int32, (_B, _B), 1)
    eye = r == c
    dist_ref[...] = jnp.full((_B, _N), 1e10, jnp.float32)

    def t_row(v):
        vb = jnp.broadcast_to(v, (_B, _B))
        return jnp.sum(jnp.where(eye, vb, jnp.zeros_like(vb)), axis=0,
                       keepdims=True)

    def body(i, far):
        m2 = lane == far
        cx = jnp.sum(jnp.where(m2, x, 0.0), axis=1, keepdims=True)
        cy = jnp.sum(jnp.where(m2, y, 0.0), axis=1, keepdims=True)
        cz = jnp.sum(jnp.where(m2, z, 0.0), axis=1, keepdims=True)
        idx_ref[pl.ds(i, 1), :] = t_row(far)
        xo_ref[pl.ds(i, 1), :] = t_row(cx)
        yo_ref[pl.ds(i, 1), :] = t_row(cy)
        zo_ref[pl.ds(i, 1), :] = t_row(cz)
        dx = x - cx
        dy = y - cy
        dz = z - cz
        d = dx * dx + dy * dy + dz * dz
        dist = jnp.minimum(dist_ref[...], d)
        dist_ref[...] = dist
        mx = jnp.max(dist, axis=1, keepdims=True)
        far_new = jnp.min(jnp.where(dist == mx, lane, _N), axis=1,
                          keepdims=True)
        return far_new

    jax.lax.fori_loop(0, _S, body, jnp.zeros((_B, 1), jnp.int32))


def _fps_call(x, y, z, interpret=False):
    return pl.pallas_call(
        _fps_body,
        out_shape=[
            jax.ShapeDtypeStruct((_S, _B), jnp.int32),
            jax.ShapeDtypeStruct((_S, _B), jnp.float32),
            jax.ShapeDtypeStruct((_S, _B), jnp.float32),
            jax.ShapeDtypeStruct((_S, _B), jnp.float32),
        ],
        scratch_shapes=[pltpu.VMEM((_B, _N), jnp.float32)],
        interpret=interpret,
    )(x, y, z)


def kernel(points):
    x = points[:, :, 0]
    y = points[:, :, 1]
    z = points[:, :, 2]
    idx, xo, yo, zo = _fps_call(x, y, z)
    xyz1_ind = idx.T
    xyz1 = jnp.stack([xo.T, yo.T, zo.T], axis=-1)
    return (xyz1_ind, xyz1)
